# baseline (speedup 1.0000x reference)
import jax
import jax.numpy as jnp
from jax import lax
from jax.experimental import pallas as pl
from jax.experimental.pallas import tpu as pltpu
from jax.experimental.pallas import tpu_sc as plsc

_NC = 2
_NS = 16
_NW = _NC * _NS
_L = 16

_D = 768
_B = 4
_S = 4096
_POS_W = _S // _NW
_HALF = 64
_SUB = 32
_NGRP = _D // _L


def _emb_kernel(x_hbm, tab_hbm, pe_hbm, out_hbm, idx_v, pe_v, buf0, buf1,
                sem0, sem1):
    wid = lax.axis_index("s") * _NC + lax.axis_index("c")
    pbase = pl.multiple_of(wid * _POS_W, _POS_W)

    for b in range(_B):
        pltpu.sync_copy(x_hbm.at[b, pl.ds(pbase, _POS_W)], idx_v.at[b])

    bufs = (buf0, buf1)
    sems = (sem0, sem1)

    def gather(g):
        h, r = g // 8, g % 8
        b, s = r // 2, r % 2
        idx = idx_v.at[b, pl.ds(h * _HALF + s * _SUB, _SUB)]
        return pltpu.async_copy(tab_hbm.at[idx], bufs[g % 2], sems[g % 2])

    cps = [gather(0), None]
    for g in range(16):
        h, r = g // 8, g % 8
        b, s = r // 2, r % 2
        if g + 1 < 16:
            cps[(g + 1) % 2] = gather(g + 1)
        if r == 0:
            pltpu.sync_copy(pe_hbm.at[pl.ds(pbase + h * _HALF, _HALF)], pe_v)
        cps[g % 2].wait()
        buf = bufs[g % 2]

        @pl.loop(0, _SUB)
        def _add(row):
            for cg in range(_NGRP):
                v = pe_v[s * _SUB + row, pl.ds(cg * _L, _L)]
                plsc.addupdate(buf.at[row, pl.ds(cg * _L, _L)], v)

        tok0 = b * _S + pbase + h * _HALF + s * _SUB
        pltpu.sync_copy(buf, out_hbm.at[pl.ds(tok0, _SUB)])


def kernel(x, tok_table, pe):
    run = pl.kernel(
        _emb_kernel,
        out_type=jax.ShapeDtypeStruct((_B * _S, _D), jnp.float32),
        mesh=plsc.VectorSubcoreMesh(core_axis_name="c", subcore_axis_name="s"),
        scratch_types=[
            pltpu.VMEM((_B, _POS_W), jnp.int32),
            pltpu.VMEM((_HALF, _D), jnp.float32),
            pltpu.VMEM((_SUB, _D), jnp.float32),
            pltpu.VMEM((_SUB, _D), jnp.float32),
            pltpu.SemaphoreType.DMA,
            pltpu.SemaphoreType.DMA,
        ],
    )
    out = run(x, tok_table, pe)
    return out.reshape(_B, _S, _D)
